# baseline (speedup 1.0000x reference)
import jax
import jax.numpy as jnp
from jax import lax
from jax.experimental import pallas as pl
from jax.experimental.pallas import tpu as pltpu

B = 64
RD = 384
H = 768
ND = 1000
K_TOTAL = 100000
KB = 12544
NBLK = (K_TOTAL + KB - 1) // KB

_NEG = float("-inf")


def _weight_copies(wp_any, wg1_any, wg2_any, wf_any, wd_any,
                   wp_s, wg1_s, wg2_s, wf_s, wd_s, wsem):
    return [
        pltpu.make_async_copy(wp_any, wp_s, wsem),
        pltpu.make_async_copy(wg1_any, wg1_s, wsem),
        pltpu.make_async_copy(wg2_any, wg2_s, wsem),
        pltpu.make_async_copy(wf_any, wf_s, wsem),
        pltpu.make_async_copy(wd_any, wd_s, wsem),
    ]


def _body(q_ref, c_ref, bn_ref, corpus_any,
          wp_any, wg1_any, wg2_any, wf_any, wd_any,
          bp_ref, bg1_ref, bg2_ref, bf_ref, g_ref, be_ref, bd_ref,
          logits_ref, gate_ref,
          rv_ref, ri_ref, ri_smem, rows_ref,
          wp_s, wg1_s, wg2_s, wf_s, wd_s, wsem, rsem, isem):
    t = pl.program_id(0)

    @pl.when(t == 0)
    def _init():
        rv_ref[...] = jnp.full((B, 128), _NEG, jnp.float32)
        ri_ref[...] = jnp.zeros((B, 128), jnp.int32)
        for c in _weight_copies(wp_any, wg1_any, wg2_any, wf_any, wd_any,
                                wp_s, wg1_s, wg2_s, wf_s, wd_s, wsem):
            c.start()

    s = lax.dot_general(q_ref[...], c_ref[...],
                        (((1,), (1,)), ((), ())),
                        preferred_element_type=jnp.float32)
    base = t * KB
    lidx = lax.broadcasted_iota(jnp.int32, (B, KB), 1)
    s = jnp.where(base + lidx < K_TOTAL, s, _NEG)

    big = jnp.int32(2 ** 30)
    cands = []
    for _ in range(3):
        m = jnp.max(s, axis=1, keepdims=True)
        i = jnp.min(jnp.where(s == m, lidx, big), axis=1, keepdims=True)
        s = jnp.where(lidx == i, _NEG, s)
        cands.append((m, i + base))

    rv = rv_ref[...]
    ri = ri_ref[...]
    v0, v1, v2 = rv[:, 0:1], rv[:, 1:2], rv[:, 2:3]
    i0, i1, i2 = ri[:, 0:1], ri[:, 1:2], ri[:, 2:3]
    for m, gi in cands:
        b0 = m > v0
        b1 = m > v1
        b2 = m > v2
        b01 = jnp.logical_or(b0, b1)
        nv0 = jnp.where(b0, m, v0)
        ni0 = jnp.where(b0, gi, i0)
        nv1 = jnp.where(b0, v0, jnp.where(b1, m, v1))
        ni1 = jnp.where(b0, i0, jnp.where(b1, gi, i1))
        nv2 = jnp.where(b01, v1, jnp.where(b2, m, v2))
        ni2 = jnp.where(b01, i1, jnp.where(b2, gi, i2))
        v0, v1, v2, i0, i1, i2 = nv0, nv1, nv2, ni0, ni1, ni2

    pad_v = jnp.full((B, 125), _NEG, jnp.float32)
    pad_i = jnp.zeros((B, 125), jnp.int32)
    rv_ref[...] = jnp.concatenate([v0, v1, v2, pad_v], axis=1)
    ri_ref[...] = jnp.concatenate([i0, i1, i2, pad_i], axis=1)

    @pl.when(t == NBLK - 1)
    def _fin():
        icopy = pltpu.make_async_copy(ri_ref, ri_smem, isem)
        icopy.start()
        icopy.wait()
        rcopies = []
        for j in range(3):
            for i in range(B):
                ridx = ri_smem[i, j]
                c = pltpu.make_async_copy(
                    corpus_any.at[pl.ds(ridx, 1)],
                    rows_ref.at[pl.ds(j * B + i, 1)], rsem)
                c.start()
                rcopies.append(c)
        for c in rcopies:
            c.wait()
        for c in _weight_copies(wp_any, wg1_any, wg2_any, wf_any, wd_any,
                                wp_s, wg1_s, wg2_s, wf_s, wd_s, wsem):
            c.wait()

        rows = rows_ref[...]
        pooled = (rows[0:B] + rows[B:2 * B] + rows[2 * B:3 * B]) \
            * jnp.float32(1.0 / 3.0)
        bn = bn_ref[...]

        def mm(a, b):
            return lax.dot_general(a, b, (((1,), (0,)), ((), ())),
                                   preferred_element_type=jnp.float32)

        rag = mm(pooled, wp_s[...]) + bp_ref[...]
        h = jnp.maximum(mm(bn, wg1_s[0:H]) + mm(rag, wg1_s[H:2 * H])
                        + bg1_ref[...], 0.0)
        glog = jnp.sum(h * wg2_s[...], axis=1, keepdims=True) + bg2_ref[0, 0]
        gate = jax.nn.sigmoid(glog)
        comb = gate * rag + (1.0 - gate) * bn
        f = mm(bn, wf_s[0:H]) + mm(comb, wf_s[H:2 * H]) + bf_ref[...]
        mu = jnp.mean(f, axis=1, keepdims=True)
        var = jnp.mean((f - mu) * (f - mu), axis=1, keepdims=True)
        f = (f - mu) / jnp.sqrt(var + 1e-5) * g_ref[...] + be_ref[...]
        f = jnp.maximum(f, 0.0)
        logits_ref[...] = mm(f, wd_s[...]) + bd_ref[...]
        gate_ref[...] = jnp.broadcast_to(gate, (B, 128))


def _call(q, corpus, bn, wp, wg1, wg2_row, wf, wd,
          bp, bg1, bg2, bf, gamma, beta, bd, interpret=False):
    return pl.pallas_call(
        _body,
        grid=(NBLK,),
        in_specs=[
            pl.BlockSpec((B, RD), lambda t: (0, 0)),
            pl.BlockSpec((KB, RD), lambda t: (t, 0)),
            pl.BlockSpec((B, H), lambda t: (0, 0)),
            pl.BlockSpec(memory_space=pl.ANY),
            pl.BlockSpec(memory_space=pl.ANY),
            pl.BlockSpec(memory_space=pl.ANY),
            pl.BlockSpec(memory_space=pl.ANY),
            pl.BlockSpec(memory_space=pl.ANY),
            pl.BlockSpec(memory_space=pl.ANY),
            pl.BlockSpec((1, H), lambda t: (0, 0)),
            pl.BlockSpec((1, H), lambda t: (0, 0)),
            pl.BlockSpec(memory_space=pltpu.SMEM),
            pl.BlockSpec((1, H), lambda t: (0, 0)),
            pl.BlockSpec((1, H), lambda t: (0, 0)),
            pl.BlockSpec((1, H), lambda t: (0, 0)),
            pl.BlockSpec((1, ND), lambda t: (0, 0)),
        ],
        out_specs=[
            pl.BlockSpec((B, ND), lambda t: (0, 0)),
            pl.BlockSpec((B, 128), lambda t: (0, 0)),
        ],
        out_shape=[
            jax.ShapeDtypeStruct((B, ND), jnp.float32),
            jax.ShapeDtypeStruct((B, 128), jnp.float32),
        ],
        scratch_shapes=[
            pltpu.VMEM((B, 128), jnp.float32),
            pltpu.VMEM((B, 128), jnp.int32),
            pltpu.SMEM((B, 128), jnp.int32),
            pltpu.VMEM((3 * B, RD), jnp.float32),
            pltpu.VMEM((RD, H), jnp.float32),
            pltpu.VMEM((2 * H, H), jnp.float32),
            pltpu.VMEM((1, H), jnp.float32),
            pltpu.VMEM((2 * H, H), jnp.float32),
            pltpu.VMEM((H, ND), jnp.float32),
            pltpu.SemaphoreType.DMA,
            pltpu.SemaphoreType.DMA,
            pltpu.SemaphoreType.DMA,
        ],
        compiler_params=pltpu.CompilerParams(
            dimension_semantics=("arbitrary",),
        ),
        interpret=interpret,
    )(q, corpus, bn, corpus, wp, wg1, wg2_row, wf, wd,
      bp, bg1, bg2, bf, gamma, beta, bd)


def kernel(bottleneck, query_emb, corpus_emb, W_proj, b_proj, W_g1, b_g1,
           W_g2, b_g2, W_f, b_f, gamma, beta, W_d, b_d):
    logits, gate128 = _call(
        query_emb, corpus_emb, bottleneck,
        W_proj, W_g1, W_g2.reshape(1, H), W_f, W_d,
        b_proj.reshape(1, H), b_g1.reshape(1, H), b_g2.reshape(1, 1),
        b_f.reshape(1, H), gamma.reshape(1, H), beta.reshape(1, H),
        b_d.reshape(1, ND))
    return logits, gate128[:, :1]
